# baseline (speedup 1.0000x reference)
import jax
import jax.numpy as jnp
from jax import lax
from jax.experimental import pallas as pl
from jax.experimental.pallas import tpu as pltpu
from jax.experimental.pallas import tpu_sc as plsc

_N = 10000
_D = 128
_E = 320000
_C = 128
_NC = 2
_NS = 16
_NW = _NC * _NS
_CPW = 80
_EPW = _CPW * _C
_ROWS = _NW * _CPW
_EPAD = _ROWS * _C
_NG = 2
_NST = 4


def _layer_body(h_hbm, e2_hbm, w_hbm, out_hbm,
                partial, gbufs, ebufs, wbufs, gsems, esems):
    zbuf = gbufs[0]
    c = lax.axis_index("c")
    s = lax.axis_index("s")
    wid = s * _NC + c
    cbase = wid * _CPW

    def _zrow(i, carry):
        for j in range(8):
            zbuf[i, pl.ds(j * 16, 16)] = jnp.zeros((16,), jnp.float32)
        return carry
    lax.fori_loop(0, _C, _zrow, 0)

    @pl.when(s < _NS - 1)
    def _zero_full():
        def _zc(k, carry):
            pltpu.sync_copy(zbuf, partial.at[pl.ds(s * 640 + k * _C, _C), :])
            return carry
        lax.fori_loop(0, 640 // _C, _zc, 0)

    @pl.when(s == _NS - 1)
    def _zero_tail():
        for k in range(400 // _C):
            pltpu.sync_copy(zbuf, partial.at[pl.ds(9600 + k * _C, _C), :])
        pltpu.sync_copy(zbuf.at[pl.ds(0, 16), :], partial.at[pl.ds(9984, 16), :])

    plsc.subcore_barrier()

    def _stage_start(k, st):
        pltpu.async_copy(e2_hbm.at[cbase + k], ebufs[st], esems[st])
        pltpu.async_copy(w_hbm.at[pl.ds((cbase + k) * _C, _C)],
                         wbufs[st].at[pl.ds(0, _C)], esems[st])

    def _stage_drain(k, st):
        pltpu.make_async_copy(e2_hbm.at[cbase + k], ebufs[st],
                              esems[st]).wait()
        pltpu.make_async_copy(w_hbm.at[pl.ds((cbase + k) * _C, _C)],
                              wbufs[st].at[pl.ds(0, _C)], esems[st]).wait()

    def _gather_start(k, st, g):
        pltpu.async_copy(h_hbm.at[ebufs[st].at[0]], gbufs[g], gsems[g])

    def _gather_drain(k, st, g):
        pltpu.make_async_copy(h_hbm.at[ebufs[st].at[0]], gbufs[g],
                              gsems[g]).wait()

    def _scale(g, st):
        def _pair(e2, carry):
            for u in range(2):
                e = 2 * e2 + u
                w = wbufs[st][pl.ds(e, 16)][0]
                for j in range(8):
                    gbufs[g][e, pl.ds(j * 16, 16)] = (
                        gbufs[g][e, pl.ds(j * 16, 16)] * w)
            return carry
        lax.fori_loop(0, _C // 2, _pair, 0)

    def _visit(k, g, st, st_next, stage_next, gather_next):
        _gather_drain(k, st, g)
        _scale(g, st)
        pltpu.sync_copy(gbufs[g], partial.at[ebufs[st].at[1]], add=True)
        if stage_next:
            _stage_start(k + _NST, st)
        if gather_next:
            _stage_drain(k + _NG, st_next)
            _gather_start(k + _NG, st_next, g)

    for k in range(_NST):
        _stage_start(k, k)
    for k in range(_NG):
        _stage_drain(k, k)
        _gather_start(k, k, k)

    def _steady(i, carry):
        for u in range(_NST):
            k = i * _NST + u
            _visit(k, u % _NG, u, (u + _NG) % _NST, True, True)
        return carry
    lax.fori_loop(0, _CPW // _NST - 1, _steady, 0)

    for u in range(_NST):
        k = _CPW - _NST + u
        _visit(k, k % _NG, k % _NST, (k + _NG) % _NST, False, k + _NG < _CPW)

    plsc.subcore_barrier()

    @pl.when(s < _NS - 1)
    def _wb_full():
        def _wc(k, carry):
            rows = pl.ds(s * 640 + k * 128, 128)
            pltpu.sync_copy(partial.at[rows, :], out_hbm.at[c, rows, :])
            return carry
        lax.fori_loop(0, 5, _wc, 0)

    @pl.when(s == _NS - 1)
    def _wb_tail():
        for k in range(3):
            rows = pl.ds(9600 + k * 128, 128)
            pltpu.sync_copy(partial.at[rows, :], out_hbm.at[c, rows, :])
        rows = pl.ds(9984, 16)
        pltpu.sync_copy(partial.at[rows, :], out_hbm.at[c, rows, :])


_layer = pl.kernel(
    _layer_body,
    out_type=jax.ShapeDtypeStruct((_NC, _N, _D), jnp.float32),
    mesh=plsc.VectorSubcoreMesh(
        core_axis_name="c", subcore_axis_name="s",
        num_cores=_NC, num_subcores=_NS),
    scratch_types=[
        pltpu.VMEM_SHARED((_N, _D), jnp.float32),
        [pltpu.VMEM((_C, _D), jnp.float32)] * _NG,
        [pltpu.VMEM((2, _C), jnp.int32)] * _NST,
        [pltpu.VMEM((_C + 16,), jnp.float32)] * _NST,
        [pltpu.SemaphoreType.DMA] * _NG,
        [pltpu.SemaphoreType.DMA] * _NST,
    ],
)


@jax.jit
def _lgcn(x, src, dst, w):
    npad = _EPAD - _E
    src2d = jnp.concatenate([src, jnp.zeros((npad,), jnp.int32)]).reshape(
        _ROWS, _C)
    dst2d = jnp.concatenate([dst, jnp.zeros((npad,), jnp.int32)]).reshape(
        _ROWS, _C)
    e2 = jnp.stack([src2d, dst2d], axis=1)
    w1d = jnp.concatenate([w, jnp.zeros((npad,), jnp.float32)])
    feats = [x]
    h = x
    for _ in range(3):
        p = _layer(h, e2, w1d)
        h = p[0] + p[1]
        feats.append(h)
    return jnp.concatenate(feats, axis=1)


def kernel(x, edge_index, edge_weight, edge_type):
    del edge_type
    src = edge_index[0].astype(jnp.int32)
    dst = edge_index[1].astype(jnp.int32)
    w = 0.5 * edge_weight.astype(jnp.float32)
    return _lgcn(x, src, dst, w)
